# Initial kernel scaffold; baseline (speedup 1.0000x reference)
import jax
import jax.numpy as jnp
from jax.experimental import pallas as pl


def kernel(x, z, Wm2, bm2, Wm3, bm3, Wv2, bv2, Wv3, bv3, Wmx, bmx, Wvx, bvx, Wmz, bmz, Wvz, bvz):
    raise NotImplementedError("write your pallas kernel here")


# speedup vs baseline: 7.3745x; 7.3745x over previous
import jax
import jax.numpy as jnp
from jax.experimental import pallas as pl
from jax.experimental.pallas import tpu as pltpu

_BATCH = 8
_NPTS = 1024
_DF = 3
_NN = 16
_R = 256
_NB = _NPTS // _R
_BIG = 1e30


def _body(x3_ref, xT_ref, z_ref,
          Wm2T_ref, bm2_ref, Wm3T_ref, bm3_ref,
          Wv2T_ref, bv2_ref, Wv3T_ref, bv3_ref,
          WmxT_ref, bmx_ref, WvxT_ref, bvx_ref,
          WmzT_ref, bmz_ref, WvzT_ref, bvz_ref,
          xout_ref, zout_ref):
    rb = pl.program_id(1)

    X0 = xT_ref[0, 0:1, :]
    X1 = xT_ref[0, 1:2, :]
    X2 = xT_ref[0, 2:3, :]
    q = x3_ref[0]
    d0 = q[:, 0:1] - X0
    d1 = q[:, 1:2] - X1
    d2 = q[:, 2:3] - X2
    D = d0 * d0 + d1 * d1 + d2 * d2
    D = jnp.where(D == 0.0, _BIG, D)

    iota = jax.lax.broadcasted_iota(jnp.int32, (_R, _NPTS), 1)
    c0s, c1s, c2s = [], [], []
    for _ in range(_NN):
        m = jnp.min(D, axis=1, keepdims=True)
        jsel = jnp.where(D == m, iota, jnp.int32(2 ** 30))
        jmin = jnp.min(jsel, axis=1, keepdims=True)
        onehot = iota == jmin
        c0s.append(jnp.sum(jnp.where(onehot, X0, 0.0), axis=1, keepdims=True))
        c1s.append(jnp.sum(jnp.where(onehot, X1, 0.0), axis=1, keepdims=True))
        c2s.append(jnp.sum(jnp.where(onehot, X2, 0.0), axis=1, keepdims=True))
        D = jnp.where(onehot, _BIG, D)

    def stack(cols):
        return jnp.concatenate(cols, axis=0)

    relu = lambda a: jnp.maximum(a, 0.0)
    third = jnp.float32(1.0 / 3.0)

    m2 = jnp.zeros((15 * _R, 32), jnp.float32)
    v2 = jnp.zeros((15 * _R, 32), jnp.float32)
    m3 = jnp.zeros((15 * _R, 32), jnp.float32)
    v3 = jnp.zeros((15 * _R, 32), jnp.float32)
    bm2 = bm2_ref[...]
    bv2 = bv2_ref[...]
    bm3 = bm3_ref[...]
    bv3 = bv3_ref[...]
    for cf in (c0s, c1s, c2s):
        t1 = stack([cf[0]] * 15)
        t2 = stack(cf[1:16])
        u2 = stack([cf[1]] * 14 + [cf[2]])
        u3 = stack(cf[2:16] + [cf[3]])
        m2 = m2 + relu(t1 * Wm2T_ref[0:1, :] + t2 * Wm2T_ref[1:2, :] + bm2)
        v2 = v2 + relu(t1 * Wv2T_ref[0:1, :] + t2 * Wv2T_ref[1:2, :] + bv2)
        m3 = m3 + relu(t1 * Wm3T_ref[0:1, :] + u2 * Wm3T_ref[1:2, :]
                       + u3 * Wm3T_ref[2:3, :] + bm3)
        v3 = v3 + relu(t1 * Wv3T_ref[0:1, :] + u2 * Wv3T_ref[1:2, :]
                       + u3 * Wv3T_ref[2:3, :] + bv3)

    moments_m = jnp.concatenate([m2 * third, m3 * third], axis=1)
    moments_v = jnp.concatenate([v2 * third, v3 * third], axis=1)

    zsc = z_ref[0, 0, 0]
    zm = zsc * WmzT_ref[...] + bmz_ref[...]
    zv = zsc * WvzT_ref[...] + bvz_ref[...]

    xm = jnp.dot(moments_m, WmxT_ref[...], preferred_element_type=jnp.float32)
    xm = relu(xm + bmx_ref[...] + zm)
    xs = xm[0:_R]
    for k in range(1, 15):
        xs = xs + xm[k * _R:(k + 1) * _R]
    xout_ref[0] = xs * jnp.float32(1.0 / 15.0)

    vm = jnp.dot(moments_v, WvxT_ref[...], preferred_element_type=jnp.float32)
    vm = relu(vm + bvx_ref[...] + zv)
    part = jnp.sum(vm, axis=0, keepdims=True)
    acc = jnp.where(rb == 0, part, zout_ref[0] + part)
    zout_ref[0] = jnp.where(rb == _NB - 1,
                            acc * jnp.float32(1.0 / (15.0 * _NPTS)), acc)


def kernel(x, z, Wm2, bm2, Wm3, bm3, Wv2, bv2, Wv3, bv3,
           Wmx, bmx, Wvx, bvx, Wmz, bmz, Wvz, bvz):
    bs = x.shape[0]
    x3 = x.reshape(bs, _NPTS, _DF)
    xT = jnp.transpose(x3, (0, 2, 1))

    row = lambda a: a.reshape(1, -1)
    grid = (bs, _NB)
    full = lambda shp: pl.BlockSpec(shp, lambda b, r: (0,) * len(shp))

    xout, zout = pl.pallas_call(
        _body,
        grid=grid,
        in_specs=[
            pl.BlockSpec((1, _R, _DF), lambda b, r: (b, r, 0)),
            pl.BlockSpec((1, _DF, _NPTS), lambda b, r: (b, 0, 0)),
            pl.BlockSpec((1, 1, 1), lambda b, r: (b, 0, 0)),
            full((2, 32)), full((1, 32)),
            full((3, 32)), full((1, 32)),
            full((2, 32)), full((1, 32)),
            full((3, 32)), full((1, 32)),
            full((64, 32)), full((1, 32)),
            full((64, 64)), full((1, 64)),
            full((1, 32)), full((1, 32)),
            full((1, 64)), full((1, 64)),
        ],
        out_specs=[
            pl.BlockSpec((1, _R, 32), lambda b, r: (b, r, 0)),
            pl.BlockSpec((1, 1, 64), lambda b, r: (b, 0, 0)),
        ],
        out_shape=[
            jax.ShapeDtypeStruct((bs, _NPTS, 32), jnp.float32),
            jax.ShapeDtypeStruct((bs, 1, 64), jnp.float32),
        ],
        compiler_params=pltpu.CompilerParams(
            dimension_semantics=("arbitrary", "arbitrary"),
        ),
    )(x3, xT, z.reshape(bs, 1, 1),
      Wm2.T, row(bm2), Wm3.T, row(bm3),
      Wv2.T, row(bv2), Wv3.T, row(bv3),
      Wmx.T, row(bmx), Wvx.T, row(bvx),
      Wmz.T, row(bmz), Wvz.T, row(bvz))

    return xout.reshape(bs, _NPTS * 32), zout.reshape(bs, 64)


# speedup vs baseline: 8.5046x; 1.1532x over previous
import jax
import jax.numpy as jnp
from jax import lax
from jax.experimental import pallas as pl
from jax.experimental.pallas import tpu as pltpu
from jax.experimental.pallas import tpu_sc as plsc

_BATCH = 8
_NPTS = 1024
_DF = 3
_NN = 16
_R = 256
_NB = _NPTS // _R
_PAD = 16
_BIG = 1e30


def _topk_body(x3_ref, xTq_ref, idx_ref):
    b = pl.program_id(0)
    X = x3_ref[0]
    Q = xTq_ref[0]
    d0 = X[:, 0:1] - Q[0:1, :]
    d1 = X[:, 1:2] - Q[1:2, :]
    d2 = X[:, 2:3] - Q[2:3, :]
    D = d0 * d0 + d1 * d1 + d2 * d2
    D = jnp.where(D == 0.0, _BIG, D)

    iota = lax.broadcasted_iota(jnp.int32, (_NPTS, _R), 0)
    rows = []
    for _ in range(_NN):
        m = jnp.min(D, axis=0, keepdims=True)
        jsel = jnp.where(D == m, iota, jnp.int32(2 ** 30))
        jmin = jnp.min(jsel, axis=0, keepdims=True)
        onehot = iota == jmin
        D = jnp.where(onehot, _BIG, D)
        rows.append(jmin)
    idx_ref[0, 0] = jnp.concatenate(rows, axis=0) + b * _NPTS


def _run_topk_tc(x3, xT):
    return pl.pallas_call(
        _topk_body,
        grid=(_BATCH, _NB),
        in_specs=[
            pl.BlockSpec((1, _NPTS, _DF), lambda b, r: (b, 0, 0)),
            pl.BlockSpec((1, _DF, _R), lambda b, r: (b, 0, r)),
        ],
        out_specs=pl.BlockSpec((1, 1, _NN, _R), lambda b, r: (b, r, 0, 0)),
        out_shape=jax.ShapeDtypeStruct((_BATCH, _NB, _NN, _R), jnp.int32),
        compiler_params=pltpu.CompilerParams(
            dimension_semantics=("arbitrary", "arbitrary"),
        ),
    )(x3, xT)


def _gather_sc(x3p_hbm, idx_hbm, out_hbm, ibuf_v, gbuf_v, semi, semg):
    wid = lax.axis_index("s") * 2 + lax.axis_index("c")
    b = wid // _NB
    rb = wid % _NB

    nseg = (_R * _NN) // 128
    hs = []
    for r in range(nseg):
        hs.append(pltpu.async_copy(
            idx_hbm.at[b, rb, r // 2, pl.ds((r % 2) * 128, 128)],
            ibuf_v.at[r], semi))
    for h in hs:
        h.wait()
    hs = []
    for r in range(nseg):
        hs.append(pltpu.async_copy(
            x3p_hbm.at[ibuf_v.at[r]],
            gbuf_v.at[pl.ds(r * 128, 128)], semg))
    for h in hs:
        h.wait()
    pltpu.sync_copy(gbuf_v, out_hbm.at[b, pl.ds(rb * _R * _NN, _R * _NN)])


def _run_gather_sc(x3p, idxg):
    mesh = plsc.VectorSubcoreMesh(core_axis_name="c", subcore_axis_name="s")
    return pl.kernel(
        _gather_sc,
        mesh=mesh,
        out_type=jax.ShapeDtypeStruct((_BATCH, _NPTS * _NN, _PAD), jnp.float32),
        scratch_types=[
            pltpu.VMEM((32, 128), jnp.int32),
            pltpu.VMEM((_R * _NN, _PAD), jnp.float32),
            pltpu.SemaphoreType.DMA,
            pltpu.SemaphoreType.DMA,
        ],
        compiler_params=pltpu.CompilerParams(use_tc_tiling_on_sc=False),
    )(x3p, idxg)


def _mlp_body(nb_ref, z_ref,
              Wm2T_ref, bm2_ref, Wm3T_ref, bm3_ref,
              Wv2T_ref, bv2_ref, Wv3T_ref, bv3_ref,
              WmxT_ref, bmx_ref, WvxT_ref, bvx_ref,
              WmzT_ref, bmz_ref, WvzT_ref, bvz_ref,
              xout_ref, zout_ref):
    rb = pl.program_id(1)
    nb4 = nb_ref[0]

    def stack(cols):
        return jnp.concatenate(cols, axis=0)

    relu = lambda a: jnp.maximum(a, 0.0)
    third = jnp.float32(1.0 / 3.0)

    m2 = jnp.zeros((15 * _R, 32), jnp.float32)
    v2 = jnp.zeros((15 * _R, 32), jnp.float32)
    m3 = jnp.zeros((15 * _R, 32), jnp.float32)
    v3 = jnp.zeros((15 * _R, 32), jnp.float32)
    bm2 = bm2_ref[...]
    bv2 = bv2_ref[...]
    bm3 = bm3_ref[...]
    bv3 = bv3_ref[...]
    for f in range(_DF):
        cf = [nb4[s, :, f:f + 1] for s in range(_NN)]
        t1 = stack([cf[0]] * 15)
        t2 = stack(cf[1:16])
        u2 = stack([cf[1]] * 14 + [cf[2]])
        u3 = stack(cf[2:16] + [cf[3]])
        m2 = m2 + relu(t1 * Wm2T_ref[0:1, :] + t2 * Wm2T_ref[1:2, :] + bm2)
        v2 = v2 + relu(t1 * Wv2T_ref[0:1, :] + t2 * Wv2T_ref[1:2, :] + bv2)
        m3 = m3 + relu(t1 * Wm3T_ref[0:1, :] + u2 * Wm3T_ref[1:2, :]
                       + u3 * Wm3T_ref[2:3, :] + bm3)
        v3 = v3 + relu(t1 * Wv3T_ref[0:1, :] + u2 * Wv3T_ref[1:2, :]
                       + u3 * Wv3T_ref[2:3, :] + bv3)

    moments_m = jnp.concatenate([m2 * third, m3 * third], axis=1)
    moments_v = jnp.concatenate([v2 * third, v3 * third], axis=1)

    zsc = z_ref[0, 0, 0]
    zm = zsc * WmzT_ref[...] + bmz_ref[...]
    zv = zsc * WvzT_ref[...] + bvz_ref[...]

    xm = jnp.dot(moments_m, WmxT_ref[...], preferred_element_type=jnp.float32)
    xm = relu(xm + bmx_ref[...] + zm)
    xs = xm[0:_R]
    for k in range(1, 15):
        xs = xs + xm[k * _R:(k + 1) * _R]
    xout_ref[0] = xs * jnp.float32(1.0 / 15.0)

    vm = jnp.dot(moments_v, WvxT_ref[...], preferred_element_type=jnp.float32)
    vm = relu(vm + bvx_ref[...] + zv)
    part = jnp.sum(vm, axis=0, keepdims=True)
    acc = jnp.where(rb == 0, part, zout_ref[0] + part)
    zout_ref[0] = jnp.where(rb == _NB - 1,
                            acc * jnp.float32(1.0 / (15.0 * _NPTS)), acc)


def kernel(x, z, Wm2, bm2, Wm3, bm3, Wv2, bv2, Wv3, bv3,
           Wmx, bmx, Wvx, bvx, Wmz, bmz, Wvz, bvz):
    bs = x.shape[0]
    x3 = x.reshape(bs, _NPTS, _DF)
    xT = jnp.transpose(x3, (0, 2, 1))

    idxg = _run_topk_tc(x3, xT)
    x3p = jnp.pad(x3.reshape(bs * _NPTS, _DF),
                  ((0, 0), (0, _PAD - _DF)))
    neigh = _run_gather_sc(x3p, idxg)
    neigh4 = neigh.reshape(bs, _NB * _NN, _R, _PAD)

    row = lambda a: a.reshape(1, -1)
    full = lambda shp: pl.BlockSpec(shp, lambda b, r: (0,) * len(shp))

    xout, zout = pl.pallas_call(
        _mlp_body,
        grid=(bs, _NB),
        in_specs=[
            pl.BlockSpec((1, _NN, _R, _PAD), lambda b, r: (b, r, 0, 0)),
            pl.BlockSpec((1, 1, 1), lambda b, r: (b, 0, 0)),
            full((2, 32)), full((1, 32)),
            full((3, 32)), full((1, 32)),
            full((2, 32)), full((1, 32)),
            full((3, 32)), full((1, 32)),
            full((64, 32)), full((1, 32)),
            full((64, 64)), full((1, 64)),
            full((1, 32)), full((1, 32)),
            full((1, 64)), full((1, 64)),
        ],
        out_specs=[
            pl.BlockSpec((1, _R, 32), lambda b, r: (b, r, 0)),
            pl.BlockSpec((1, 1, 64), lambda b, r: (b, 0, 0)),
        ],
        out_shape=[
            jax.ShapeDtypeStruct((bs, _NPTS, 32), jnp.float32),
            jax.ShapeDtypeStruct((bs, 1, 64), jnp.float32),
        ],
        compiler_params=pltpu.CompilerParams(
            dimension_semantics=("arbitrary", "arbitrary"),
        ),
    )(neigh4, z.reshape(bs, 1, 1),
      Wm2.T, row(bm2), Wm3.T, row(bm3),
      Wv2.T, row(bv2), Wv3.T, row(bv3),
      Wmx.T, row(bmx), Wvx.T, row(bvx),
      Wmz.T, row(bmz), Wvz.T, row(bvz))

    return xout.reshape(bs, _NPTS * 32), zout.reshape(bs, 64)


# speedup vs baseline: 10.4600x; 1.2299x over previous
import jax
import jax.numpy as jnp
from jax import lax
from jax.experimental import pallas as pl
from jax.experimental.pallas import tpu as pltpu
from jax.experimental.pallas import tpu_sc as plsc

_BATCH = 8
_NPTS = 1024
_DF = 3
_NN = 16
_R = 256
_NB = _NPTS // _R
_PAD = 16
_BIG = 1e30


def _topk_body(x3_ref, xTq_ref, idx_ref):
    b = pl.program_id(0)
    X = x3_ref[0]
    Q = xTq_ref[0]
    d0 = X[:, 0:1] - Q[0:1, :]
    d1 = X[:, 1:2] - Q[1:2, :]
    d2 = X[:, 2:3] - Q[2:3, :]
    D = d0 * d0 + d1 * d1 + d2 * d2
    D = jnp.where(D == 0.0, _BIG, D)

    iota = lax.broadcasted_iota(jnp.int32, (_NPTS, _R), 0)
    rows = []
    for _ in range(_NN):
        m = jnp.min(D, axis=0, keepdims=True)
        jsel = jnp.where(D == m, iota, jnp.int32(2 ** 30))
        jmin = jnp.min(jsel, axis=0, keepdims=True)
        onehot = iota == jmin
        D = jnp.where(onehot, _BIG, D)
        rows.append(jmin)
    idx_ref[0, 0] = jnp.concatenate(rows, axis=0) + b * _NPTS


def _run_topk_tc(x3, xT):
    return pl.pallas_call(
        _topk_body,
        grid=(_BATCH, _NB),
        in_specs=[
            pl.BlockSpec((1, _NPTS, _DF), lambda b, r: (b, 0, 0)),
            pl.BlockSpec((1, _DF, _R), lambda b, r: (b, 0, r)),
        ],
        out_specs=pl.BlockSpec((1, 1, _NN, _R), lambda b, r: (b, r, 0, 0)),
        out_shape=jax.ShapeDtypeStruct((_BATCH, _NB, _NN, _R), jnp.int32),
        compiler_params=pltpu.CompilerParams(
            dimension_semantics=("arbitrary", "arbitrary"),
        ),
    )(x3, xT)


def _gather_sc(x3p_hbm, idx_hbm, out_hbm, ibuf_v, gbuf_v, semi, semg):
    wid = lax.axis_index("s") * 2 + lax.axis_index("c")
    b = wid // _NB
    rb = wid % _NB

    nseg = (_R * _NN) // 128
    hs = []
    for r in range(nseg):
        hs.append(pltpu.async_copy(
            idx_hbm.at[b, rb, r // 2, pl.ds((r % 2) * 128, 128)],
            ibuf_v.at[r], semi))
    for h in hs:
        h.wait()
    hs = []
    for r in range(nseg):
        hs.append(pltpu.async_copy(
            x3p_hbm.at[ibuf_v.at[r]],
            gbuf_v.at[pl.ds(r * 128, 128)], semg))
    for h in hs:
        h.wait()
    pltpu.sync_copy(gbuf_v, out_hbm.at[b, pl.ds(rb * _R * _NN, _R * _NN)])


def _run_gather_sc(x3p, idxg):
    mesh = plsc.VectorSubcoreMesh(core_axis_name="c", subcore_axis_name="s")
    return pl.kernel(
        _gather_sc,
        mesh=mesh,
        out_type=jax.ShapeDtypeStruct((_BATCH, _NPTS * _NN, _PAD), jnp.float32),
        scratch_types=[
            pltpu.VMEM((32, 128), jnp.int32),
            pltpu.VMEM((_R * _NN, _PAD), jnp.float32),
            pltpu.SemaphoreType.DMA,
            pltpu.SemaphoreType.DMA,
        ],
        compiler_params=pltpu.CompilerParams(use_tc_tiling_on_sc=False),
    )(x3p, idxg)


def _mlp_body(nb_ref, z_ref, A_ref, B_ref, C_ref, E_ref, b4_ref,
              Wcat_ref, bcat_ref, Wzc_ref, bzc_ref,
              xout_ref, zout_ref):
    rb = pl.program_id(1)
    nb4 = nb_ref[0]

    def stack(cols):
        return jnp.concatenate(cols, axis=0)

    relu = lambda a: jnp.maximum(a, 0.0)
    third = jnp.float32(1.0 / 3.0)

    A = A_ref[...]
    B = B_ref[...]
    C = C_ref[...]
    E = E_ref[...]
    b4 = b4_ref[...]
    F = jnp.zeros((15 * _R, 128), jnp.float32)
    for f in range(_DF):
        cf = [nb4[s, :, f:f + 1] for s in range(_NN)]
        t1 = stack([cf[0]] * 15)
        t2 = stack(cf[1:16])
        u2 = stack([cf[1]] * 14 + [cf[2]])
        u3 = stack(cf[2:16] + [cf[3]])
        F = F + relu(t1 * A + t2 * B + u2 * C + u3 * E + b4)

    zsc = z_ref[0, 0, 0]
    zcat = zsc * Wzc_ref[...] + bzc_ref[...]

    XV = jnp.dot(F * third, Wcat_ref[...], preferred_element_type=jnp.float32)
    XV = relu(XV + bcat_ref[...] + zcat)

    xm = XV[:, 0:32]
    xs = xm[0:_R]
    for k in range(1, 15):
        xs = xs + xm[k * _R:(k + 1) * _R]
    xout_ref[0] = xs * jnp.float32(1.0 / 15.0)

    vm = XV[:, 32:96]
    part = jnp.sum(vm, axis=0, keepdims=True)
    acc = jnp.where(rb == 0, part, zout_ref[0] + part)
    zout_ref[0] = jnp.where(rb == _NB - 1,
                            acc * jnp.float32(1.0 / (15.0 * _NPTS)), acc)


def kernel(x, z, Wm2, bm2, Wm3, bm3, Wv2, bv2, Wv3, bv3,
           Wmx, bmx, Wvx, bvx, Wmz, bmz, Wvz, bvz):
    bs = x.shape[0]
    x3 = x.reshape(bs, _NPTS, _DF)
    xT = jnp.transpose(x3, (0, 2, 1))

    idxg = _run_topk_tc(x3, xT)
    x3p = jnp.pad(x3.reshape(bs * _NPTS, _DF),
                  ((0, 0), (0, _PAD - _DF)))
    neigh = _run_gather_sc(x3p, idxg)
    neigh4 = neigh.reshape(bs, _NB * _NN, _R, _PAD)

    row = lambda a: a.reshape(1, -1)
    z32 = jnp.zeros((32,), jnp.float32)
    cat = lambda parts: jnp.concatenate(parts).reshape(1, 128)
    A_row = cat([Wm2[:, 0], Wm3[:, 0], Wv2[:, 0], Wv3[:, 0]])
    B_row = cat([Wm2[:, 1], z32, Wv2[:, 1], z32])
    C_row = cat([z32, Wm3[:, 1], z32, Wv3[:, 1]])
    E_row = cat([z32, Wm3[:, 2], z32, Wv3[:, 2]])
    b4_row = cat([bm2, bm3, bv2, bv3])
    Wcat = jnp.zeros((128, 96), jnp.float32)
    Wcat = Wcat.at[0:64, 0:32].set(Wmx.T).at[64:128, 32:96].set(Wvx.T)
    bcat = jnp.concatenate([bmx, bvx]).reshape(1, 96)
    Wzc = jnp.concatenate([Wmz.T, Wvz.T], axis=1)
    bzc = jnp.concatenate([bmz, bvz]).reshape(1, 96)
    full = lambda shp: pl.BlockSpec(shp, lambda b, r: (0,) * len(shp))

    xout, zout = pl.pallas_call(
        _mlp_body,
        grid=(bs, _NB),
        in_specs=[
            pl.BlockSpec((1, _NN, _R, _PAD), lambda b, r: (b, r, 0, 0)),
            pl.BlockSpec((1, 1, 1), lambda b, r: (b, 0, 0)),
            full((1, 128)), full((1, 128)), full((1, 128)), full((1, 128)),
            full((1, 128)),
            full((128, 96)), full((1, 96)),
            full((1, 96)), full((1, 96)),
        ],
        out_specs=[
            pl.BlockSpec((1, _R, 32), lambda b, r: (b, r, 0)),
            pl.BlockSpec((1, 1, 64), lambda b, r: (b, 0, 0)),
        ],
        out_shape=[
            jax.ShapeDtypeStruct((bs, _NPTS, 32), jnp.float32),
            jax.ShapeDtypeStruct((bs, 1, 64), jnp.float32),
        ],
        compiler_params=pltpu.CompilerParams(
            dimension_semantics=("arbitrary", "arbitrary"),
        ),
    )(neigh4, z.reshape(bs, 1, 1),
      A_row, B_row, C_row, E_row, b4_row, Wcat, bcat, Wzc, bzc)

    return xout.reshape(bs, _NPTS * 32), zout.reshape(bs, 64)


# speedup vs baseline: 10.6195x; 1.0152x over previous
import jax
import jax.numpy as jnp
from jax import lax
from jax.experimental import pallas as pl
from jax.experimental.pallas import tpu as pltpu
from jax.experimental.pallas import tpu_sc as plsc

_BATCH = 8
_NPTS = 1024
_DF = 3
_NN = 16
_R = 256
_NB = _NPTS // _R
_PAD = 16
_BIG = 1e30


def _topk_body(x3_ref, xTq_ref, idx_ref, x3p_ref):
    b = pl.program_id(0)
    X = x3_ref[0]
    Q = xTq_ref[0]
    d0 = X[:, 0:1] - Q[0:1, :]
    d1 = X[:, 1:2] - Q[1:2, :]
    d2 = X[:, 2:3] - Q[2:3, :]
    D = d0 * d0 + d1 * d1 + d2 * d2
    D = jnp.where(D == 0.0, _BIG, D)

    iota = lax.broadcasted_iota(jnp.int32, (_NPTS, _R), 0)
    rows = []
    for s in range(_NN):
        m = jnp.min(D, axis=0, keepdims=True)
        jsel = jnp.where(D == m, iota, jnp.int32(2 ** 30))
        jmin = jnp.min(jsel, axis=0, keepdims=True)
        rows.append(jmin)
        if s < _NN - 1:
            D = jnp.where(jsel == jmin, _BIG, D)
    idx_ref[0, 0] = jnp.concatenate(rows, axis=0) + b * _NPTS

    @pl.when(pl.program_id(1) == 0)
    def _pad_rows():
        x3p_ref[...] = jnp.concatenate(
            [X, jnp.zeros((_NPTS, _PAD - _DF), jnp.float32)], axis=1)


def _run_topk_tc(x3, xT):
    return pl.pallas_call(
        _topk_body,
        grid=(_BATCH, _NB),
        in_specs=[
            pl.BlockSpec((1, _NPTS, _DF), lambda b, r: (b, 0, 0)),
            pl.BlockSpec((1, _DF, _R), lambda b, r: (b, 0, r)),
        ],
        out_specs=[
            pl.BlockSpec((1, 1, _NN, _R), lambda b, r: (b, r, 0, 0)),
            pl.BlockSpec((_NPTS, _PAD), lambda b, r: (b, 0)),
        ],
        out_shape=[
            jax.ShapeDtypeStruct((_BATCH, _NB, _NN, _R), jnp.int32),
            jax.ShapeDtypeStruct((_BATCH * _NPTS, _PAD), jnp.float32),
        ],
        compiler_params=pltpu.CompilerParams(
            dimension_semantics=("arbitrary", "arbitrary"),
        ),
    )(x3, xT)


def _gather_sc(x3p_hbm, idx_hbm, out_hbm, ibuf_v, gbuf_v, semi, semg):
    wid = lax.axis_index("s") * 2 + lax.axis_index("c")
    b = wid // _NB
    rb = wid % _NB

    nseg = (_R * _NN) // 128
    hs = []
    for r in range(nseg):
        hs.append(pltpu.async_copy(
            idx_hbm.at[b, rb, r // 2, pl.ds((r % 2) * 128, 128)],
            ibuf_v.at[r], semi))
    for h in hs:
        h.wait()
    hs = []
    for r in range(nseg):
        hs.append(pltpu.async_copy(
            x3p_hbm.at[ibuf_v.at[r]],
            gbuf_v.at[pl.ds(r * 128, 128)], semg))
    for h in hs:
        h.wait()
    pltpu.sync_copy(gbuf_v, out_hbm.at[b, pl.ds(rb * _R * _NN, _R * _NN)])


def _run_gather_sc(x3p, idxg):
    mesh = plsc.VectorSubcoreMesh(core_axis_name="c", subcore_axis_name="s")
    return pl.kernel(
        _gather_sc,
        mesh=mesh,
        out_type=jax.ShapeDtypeStruct((_BATCH, _NPTS * _NN, _PAD), jnp.float32),
        scratch_types=[
            pltpu.VMEM((32, 128), jnp.int32),
            pltpu.VMEM((_R * _NN, _PAD), jnp.float32),
            pltpu.SemaphoreType.DMA,
            pltpu.SemaphoreType.DMA,
        ],
        compiler_params=pltpu.CompilerParams(use_tc_tiling_on_sc=False),
    )(x3p, idxg)


def _mlp_body(nb_ref, z_ref, A_ref, B_ref, C_ref, E_ref, b4_ref,
              Wcat_ref, bcat_ref, Wzc_ref, bzc_ref,
              xout_ref, zout_ref):
    rb = pl.program_id(1)
    nb4 = nb_ref[0]

    def stack(cols):
        return jnp.concatenate(cols, axis=0)

    relu = lambda a: jnp.maximum(a, 0.0)
    third = jnp.float32(1.0 / 3.0)

    A = A_ref[...]
    B = B_ref[...]
    C = C_ref[...]
    E = E_ref[...]
    b4 = b4_ref[...]
    F = jnp.zeros((15 * _R, 128), jnp.float32)
    for f in range(_DF):
        cf = [nb4[s, :, f:f + 1] for s in range(_NN)]
        t1 = stack([cf[0]] * 15)
        t2 = stack(cf[1:16])
        u2 = stack([cf[1]] * 14 + [cf[2]])
        u3 = stack(cf[2:16] + [cf[3]])
        F = F + relu(t1 * A + t2 * B + u2 * C + u3 * E + b4)

    zsc = z_ref[0, 0, 0]
    zcat = zsc * Wzc_ref[...] + bzc_ref[...]

    XV = jnp.dot(F * third, Wcat_ref[...], preferred_element_type=jnp.float32)
    XV = relu(XV + bcat_ref[...] + zcat)

    xm = XV[:, 0:32]
    xs = xm[0:_R]
    for k in range(1, 15):
        xs = xs + xm[k * _R:(k + 1) * _R]
    xout_ref[0] = xs * jnp.float32(1.0 / 15.0)

    vm = XV[:, 32:96]
    part = jnp.sum(vm, axis=0, keepdims=True)
    acc = jnp.where(rb == 0, part, zout_ref[0] + part)
    zout_ref[0] = jnp.where(rb == _NB - 1,
                            acc * jnp.float32(1.0 / (15.0 * _NPTS)), acc)


def kernel(x, z, Wm2, bm2, Wm3, bm3, Wv2, bv2, Wv3, bv3,
           Wmx, bmx, Wvx, bvx, Wmz, bmz, Wvz, bvz):
    bs = x.shape[0]
    x3 = x.reshape(bs, _NPTS, _DF)
    xT = jnp.transpose(x3, (0, 2, 1))

    idxg, x3p = _run_topk_tc(x3, xT)
    neigh = _run_gather_sc(x3p, idxg)
    neigh4 = neigh.reshape(bs, _NB * _NN, _R, _PAD)

    row = lambda a: a.reshape(1, -1)
    z32 = jnp.zeros((32,), jnp.float32)
    cat = lambda parts: jnp.concatenate(parts).reshape(1, 128)
    A_row = cat([Wm2[:, 0], Wm3[:, 0], Wv2[:, 0], Wv3[:, 0]])
    B_row = cat([Wm2[:, 1], z32, Wv2[:, 1], z32])
    C_row = cat([z32, Wm3[:, 1], z32, Wv3[:, 1]])
    E_row = cat([z32, Wm3[:, 2], z32, Wv3[:, 2]])
    b4_row = cat([bm2, bm3, bv2, bv3])
    Wcat = jnp.zeros((128, 96), jnp.float32)
    Wcat = Wcat.at[0:64, 0:32].set(Wmx.T).at[64:128, 32:96].set(Wvx.T)
    bcat = jnp.concatenate([bmx, bvx]).reshape(1, 96)
    Wzc = jnp.concatenate([Wmz.T, Wvz.T], axis=1)
    bzc = jnp.concatenate([bmz, bvz]).reshape(1, 96)
    full = lambda shp: pl.BlockSpec(shp, lambda b, r: (0,) * len(shp))

    xout, zout = pl.pallas_call(
        _mlp_body,
        grid=(bs, _NB),
        in_specs=[
            pl.BlockSpec((1, _NN, _R, _PAD), lambda b, r: (b, r, 0, 0)),
            pl.BlockSpec((1, 1, 1), lambda b, r: (b, 0, 0)),
            full((1, 128)), full((1, 128)), full((1, 128)), full((1, 128)),
            full((1, 128)),
            full((128, 96)), full((1, 96)),
            full((1, 96)), full((1, 96)),
        ],
        out_specs=[
            pl.BlockSpec((1, _R, 32), lambda b, r: (b, r, 0)),
            pl.BlockSpec((1, 1, 64), lambda b, r: (b, 0, 0)),
        ],
        out_shape=[
            jax.ShapeDtypeStruct((bs, _NPTS, 32), jnp.float32),
            jax.ShapeDtypeStruct((bs, 1, 64), jnp.float32),
        ],
        compiler_params=pltpu.CompilerParams(
            dimension_semantics=("arbitrary", "arbitrary"),
        ),
    )(neigh4, z.reshape(bs, 1, 1),
      A_row, B_row, C_row, E_row, b4_row, Wcat, bcat, Wzc, bzc)

    return xout.reshape(bs, _NPTS * 32), zout.reshape(bs, 64)
